# baseline (speedup 1.0000x reference)
import jax
import jax.numpy as jnp
from jax.experimental import pallas as pl
from jax.experimental.pallas import tpu as pltpu

_NUM_EXPERTS = 64
_TOP_K = 8
_HIDDEN = 4096
_TOKENS = 16384
_TB = 512


def _router_body(x_ref, w_ref, logits_ref, weights_ref, ids_ref):
    x = x_ref[...]
    w = w_ref[...]
    logits_t = jax.lax.dot_general(
        w, x, (((1,), (1,)), ((), ())),
        preferred_element_type=jnp.float32)
    logits_ref[...] = logits_t.T

    inv_row = jnp.int32(_NUM_EXPERTS - 1) - jax.lax.broadcasted_iota(
        jnp.int32, logits_t.shape, 0)
    y = jax.lax.bitcast_convert_type(logits_t, jnp.int32)
    key = y ^ (jax.lax.shift_right_arithmetic(y, 31) & jnp.int32(0x7FFFFFFF))

    neg_inf_key = jnp.int32(-2147483648)
    vals = []
    idxs = []
    for _ in range(_TOP_K):
        wmax = jnp.max(key, axis=0, keepdims=True)
        cand = jnp.where(key == wmax, inv_row, jnp.int32(-1))
        wrow = jnp.max(cand, axis=0, keepdims=True)
        idxs.append(jnp.int32(_NUM_EXPERTS - 1) - wrow)
        yb = wmax ^ (jax.lax.shift_right_arithmetic(wmax, 31)
                     & jnp.int32(0x7FFFFFFF))
        vals.append(jax.lax.bitcast_convert_type(yb, jnp.float32))
        key = jnp.where(cand == wrow, neg_inf_key, key)

    topv = jnp.concatenate(vals, axis=0)
    topi = jnp.concatenate(idxs, axis=0)
    e = jnp.exp(topv - topv[:1, :])
    wts = e / jnp.sum(e, axis=0, keepdims=True)
    weights_ref[...] = wts.T
    ids_ref[...] = topi.T


def kernel(hidden_states, gate_w):
    grid = (_TOKENS // _TB,)
    out_shape = (
        jax.ShapeDtypeStruct((_TOKENS, _NUM_EXPERTS), jnp.float32),
        jax.ShapeDtypeStruct((_TOKENS, _TOP_K), jnp.float32),
        jax.ShapeDtypeStruct((_TOKENS, _TOP_K), jnp.int32),
    )
    logits, weights, ids = pl.pallas_call(
        _router_body,
        grid=grid,
        in_specs=[
            pl.BlockSpec((_TB, _HIDDEN), lambda i: (i, 0)),
            pl.BlockSpec((_NUM_EXPERTS, _HIDDEN), lambda i: (0, 0)),
        ],
        out_specs=(
            pl.BlockSpec((_TB, _NUM_EXPERTS), lambda i: (i, 0)),
            pl.BlockSpec((_TB, _TOP_K), lambda i: (i, 0)),
            pl.BlockSpec((_TB, _TOP_K), lambda i: (i, 0)),
        ),
        out_shape=out_shape,
        compiler_params=pltpu.CompilerParams(
            dimension_semantics=("parallel",),
        ),
    )(hidden_states, gate_w)
    return weights, ids, logits


# speedup vs baseline: 1.0694x; 1.0694x over previous
import jax
import jax.numpy as jnp
from jax.experimental import pallas as pl
from jax.experimental.pallas import tpu as pltpu

_NUM_EXPERTS = 64
_TOP_K = 8
_HIDDEN = 4096
_TOKENS = 16384
_TB = 1024


def _router_body(x_ref, w_ref, logits_ref, weights_ref, ids_ref):
    x = x_ref[...]
    w = w_ref[...]
    logits_t = jax.lax.dot_general(
        w, x, (((1,), (1,)), ((), ())),
        preferred_element_type=jnp.float32)
    logits_ref[...] = logits_t.T

    inv_row = jnp.int32(_NUM_EXPERTS - 1) - jax.lax.broadcasted_iota(
        jnp.int32, logits_t.shape, 0)
    y = jax.lax.bitcast_convert_type(logits_t, jnp.int32)
    key = y ^ (jax.lax.shift_right_arithmetic(y, 31) & jnp.int32(0x7FFFFFFF))

    neg_inf_key = jnp.int32(-2147483648)
    vals = []
    idxs = []
    for _ in range(_TOP_K):
        wmax = jnp.max(key, axis=0, keepdims=True)
        cand = jnp.where(key == wmax, inv_row, jnp.int32(-1))
        wrow = jnp.max(cand, axis=0, keepdims=True)
        idxs.append(jnp.int32(_NUM_EXPERTS - 1) - wrow)
        yb = wmax ^ (jax.lax.shift_right_arithmetic(wmax, 31)
                     & jnp.int32(0x7FFFFFFF))
        vals.append(jax.lax.bitcast_convert_type(yb, jnp.float32))
        key = jnp.where(cand == wrow, neg_inf_key, key)

    topv = jnp.concatenate(vals, axis=0)
    topi = jnp.concatenate(idxs, axis=0)
    e = jnp.exp(topv - topv[:1, :])
    wts = e / jnp.sum(e, axis=0, keepdims=True)
    weights_ref[...] = wts.T
    ids_ref[...] = topi.T


def kernel(hidden_states, gate_w):
    grid = (_TOKENS // _TB,)
    out_shape = (
        jax.ShapeDtypeStruct((_TOKENS, _NUM_EXPERTS), jnp.float32),
        jax.ShapeDtypeStruct((_TOKENS, _TOP_K), jnp.float32),
        jax.ShapeDtypeStruct((_TOKENS, _TOP_K), jnp.int32),
    )
    logits, weights, ids = pl.pallas_call(
        _router_body,
        grid=grid,
        in_specs=[
            pl.BlockSpec((_TB, _HIDDEN), lambda i: (i, 0)),
            pl.BlockSpec((_NUM_EXPERTS, _HIDDEN), lambda i: (0, 0)),
        ],
        out_specs=(
            pl.BlockSpec((_TB, _NUM_EXPERTS), lambda i: (i, 0)),
            pl.BlockSpec((_TB, _TOP_K), lambda i: (i, 0)),
            pl.BlockSpec((_TB, _TOP_K), lambda i: (i, 0)),
        ),
        out_shape=out_shape,
        compiler_params=pltpu.CompilerParams(
            dimension_semantics=("parallel",),
        ),
    )(hidden_states, gate_w)
    return weights, ids, logits
